# Initial kernel scaffold; baseline (speedup 1.0000x reference)
import jax
import jax.numpy as jnp
from jax.experimental import pallas as pl


def kernel(node_x, text_x, edge_index, Wl1, Wr1, b1, Wl2, Wr2, b2, Wl3, Wr3, b3, Wt1, bt1, Wt2, bt2, W4, b4):
    raise NotImplementedError("write your pallas kernel here")


# speedup vs baseline: 3.8122x; 3.8122x over previous
import functools

import jax
import jax.numpy as jnp
from jax import lax
from jax.experimental import pallas as pl
from jax.experimental.pallas import tpu as pltpu
from jax.experimental.pallas import tpu_sc as plsc

N = 10000
E = 320000
NC = 2
NS = 16
B_E = 128
E_PAD = 323584
N_ACC = 10112
PAD_DST = 10016
ROWS_PER_TILE = N_ACC // NS
NBLK = 1000


def _make_segsum(mode):
  w = 128
  e_per_tile = E_PAD // (NS * NC) if mode == "edge_split" else E_PAD // NS
  n_batches = e_per_tile // B_E
  mesh = plsc.VectorSubcoreMesh(core_axis_name="c", subcore_axis_name="s")

  @functools.partial(
      pl.kernel,
      mesh=mesh,
      out_type=jax.ShapeDtypeStruct((NC, N_ACC, w), jnp.float32),
      scratch_types=[
          pltpu.VMEM((B_E,), jnp.int32),
          pltpu.VMEM((B_E,), jnp.int32),
          pltpu.VMEM((B_E, w), jnp.float32),
          pltpu.VMEM_SHARED((N_ACC, w), jnp.float32),
          pltpu.SemaphoreType.DMA,
      ],
  )
  def seg_kernel(x_hbm, src_hbm, dst_hbm, zero_hbm, ones_hbm, out_hbm,
                 src_v, dst_v, rows_v, acc_sh, sem):
    c = lax.axis_index("c")
    s = lax.axis_index("s")
    base_r = s * ROWS_PER_TILE
    pltpu.sync_copy(zero_hbm.at[pl.ds(base_r, ROWS_PER_TILE)],
                    acc_sh.at[pl.ds(base_r, ROWS_PER_TILE)])
    plsc.subcore_barrier()

    if mode == "edge_split":
      tile_e0 = (c * NS + s) * e_per_tile
    else:
      tile_e0 = s * e_per_tile

    def gather_body(i, carry):
      e0 = tile_e0 + i * B_E
      pltpu.sync_copy(src_hbm.at[pl.ds(e0, B_E)], src_v)
      pltpu.sync_copy(dst_hbm.at[pl.ds(e0, B_E)], dst_v)
      if mode == "col_split":
        pltpu.async_copy(x_hbm.at[c].at[src_v], rows_v, sem).wait()
      else:
        pltpu.async_copy(x_hbm.at[src_v], rows_v, sem).wait()
      pltpu.sync_copy(rows_v, acc_sh.at[dst_v], add=True)
      return carry

    def ones_body(i, carry):
      e0 = tile_e0 + i * B_E
      pltpu.sync_copy(dst_hbm.at[pl.ds(e0, B_E)], dst_v)
      pltpu.sync_copy(rows_v, acc_sh.at[dst_v], add=True)
      return carry

    if mode == "l1":
      @pl.when(c == 0)
      def _():
        lax.fori_loop(0, n_batches, gather_body, 0)

      @pl.when(c == 1)
      def _():
        pltpu.sync_copy(ones_hbm, rows_v)
        lax.fori_loop(0, n_batches, ones_body, 0)
    else:
      lax.fori_loop(0, n_batches, gather_body, 0)

    plsc.subcore_barrier()
    pltpu.sync_copy(acc_sh.at[pl.ds(base_r, ROWS_PER_TILE)],
                    out_hbm.at[c].at[pl.ds(base_r, ROWS_PER_TILE)])

  return seg_kernel


_seg_l1 = _make_segsum("l1")
_seg_l2 = _make_segsum("col_split")
_seg_l3 = _make_segsum("edge_split")


def _full(shape):
  return pl.BlockSpec(shape, lambda i: tuple(0 for _ in shape))


def _text_body(x_ref, w1_ref, b1_ref, w2_ref, b2_ref, o_ref):
  h = jnp.maximum(
      jnp.dot(x_ref[...], w1_ref[...], preferred_element_type=jnp.float32)
      + b1_ref[...], 0.0)
  o_ref[...] = (jnp.dot(h, w2_ref[...], preferred_element_type=jnp.float32)
                + b2_ref[...])


def _text_call(text_x, wt1, bt1, wt2, bt2):
  grid = (N // NBLK,)
  return pl.pallas_call(
      _text_body,
      grid=grid,
      in_specs=[
          pl.BlockSpec((NBLK, 256), lambda i: (i, 0)),
          _full((256, 256)), _full((1, 256)),
          _full((256, 128)), _full((1, 128)),
      ],
      out_specs=pl.BlockSpec((NBLK, 128), lambda i: (i, 0)),
      out_shape=jax.ShapeDtypeStruct((N, 128), jnp.float32),
  )(text_x, wt1, bt1, wt2, bt2)


def _ta_body(agg_ref, x_ref, wl_ref, wr_ref, b_ref, h1_ref, inv_ref):
  inv = 1.0 / jnp.maximum(agg_ref[1][:, 0:1], 1.0)
  mean = agg_ref[0] * inv
  h1 = jnp.maximum(
      jnp.dot(mean, wl_ref[...], preferred_element_type=jnp.float32)
      + b_ref[...]
      + jnp.dot(x_ref[...], wr_ref[...], preferred_element_type=jnp.float32),
      0.0)
  h1_ref[0] = h1[:, :128]
  h1_ref[1] = h1[:, 128:]
  inv_ref[...] = inv


def _ta_call(agg1, node_x, wl1, wr1, b1):
  grid = (N // NBLK,)
  return pl.pallas_call(
      _ta_body,
      grid=grid,
      in_specs=[
          pl.BlockSpec((2, NBLK, 128), lambda i: (0, i, 0)),
          pl.BlockSpec((NBLK, 128), lambda i: (i, 0)),
          _full((128, 256)), _full((128, 256)), _full((1, 256)),
      ],
      out_specs=[
          pl.BlockSpec((2, NBLK, 128), lambda i: (0, i, 0)),
          pl.BlockSpec((NBLK, 1), lambda i: (i, 0)),
      ],
      out_shape=[
          jax.ShapeDtypeStruct((2, N, 128), jnp.float32),
          jax.ShapeDtypeStruct((N, 1), jnp.float32),
      ],
  )(agg1, node_x, wl1, wr1, b1)


def _tb_body(agg_ref, h1_ref, inv_ref, wl_ref, wr_ref, b_ref, wl3_ref,
             h2_ref, z3_ref):
  inv = inv_ref[...]
  mean = jnp.concatenate([agg_ref[0], agg_ref[1]], axis=1) * inv
  h1f = jnp.concatenate([h1_ref[0], h1_ref[1]], axis=1)
  h2 = jnp.maximum(
      jnp.dot(mean, wl_ref[...], preferred_element_type=jnp.float32)
      + b_ref[...]
      + jnp.dot(h1f, wr_ref[...], preferred_element_type=jnp.float32),
      0.0)
  h2_ref[...] = h2
  z3_ref[...] = jnp.dot(h2, wl3_ref[...], preferred_element_type=jnp.float32)


def _tb_call(agg2, h1h, inv, wl2, wr2, b2, wl3):
  grid = (N // NBLK,)
  return pl.pallas_call(
      _tb_body,
      grid=grid,
      in_specs=[
          pl.BlockSpec((2, NBLK, 128), lambda i: (0, i, 0)),
          pl.BlockSpec((2, NBLK, 128), lambda i: (0, i, 0)),
          pl.BlockSpec((NBLK, 1), lambda i: (i, 0)),
          _full((256, 256)), _full((256, 256)), _full((1, 256)),
          _full((256, 128)),
      ],
      out_specs=[
          pl.BlockSpec((NBLK, 256), lambda i: (i, 0)),
          pl.BlockSpec((NBLK, 128), lambda i: (i, 0)),
      ],
      out_shape=[
          jax.ShapeDtypeStruct((N, 256), jnp.float32),
          jax.ShapeDtypeStruct((N, 128), jnp.float32),
      ],
  )(agg2, h1h, inv, wl2, wr2, b2, wl3)


def _tc_body(agg_ref, h2_ref, txt_ref, inv_ref, wr3_ref, b3_ref,
             w4a_ref, w4b_ref, b4_ref, lbl_ref, fin_ref):
  mean = (agg_ref[0] + agg_ref[1]) * inv_ref[...]
  lbl = jnp.maximum(
      mean + b3_ref[...]
      + jnp.dot(h2_ref[...], wr3_ref[...], preferred_element_type=jnp.float32),
      0.0)
  lbl_ref[...] = lbl
  fin_ref[...] = (
      jnp.dot(lbl, w4a_ref[...], preferred_element_type=jnp.float32)
      + jnp.dot(txt_ref[...], w4b_ref[...], preferred_element_type=jnp.float32)
      + b4_ref[...])


def _tc_call(agg3, h2, txt_emb, inv, wr3, b3, w4a, w4b, b4):
  grid = (N // NBLK,)
  return pl.pallas_call(
      _tc_body,
      grid=grid,
      in_specs=[
          pl.BlockSpec((2, NBLK, 128), lambda i: (0, i, 0)),
          pl.BlockSpec((NBLK, 256), lambda i: (i, 0)),
          pl.BlockSpec((NBLK, 128), lambda i: (i, 0)),
          pl.BlockSpec((NBLK, 1), lambda i: (i, 0)),
          _full((256, 128)), _full((1, 128)),
          _full((128, 256)), _full((128, 256)), _full((1, 256)),
      ],
      out_specs=[
          pl.BlockSpec((NBLK, 128), lambda i: (i, 0)),
          pl.BlockSpec((NBLK, 256), lambda i: (i, 0)),
      ],
      out_shape=[
          jax.ShapeDtypeStruct((N, 128), jnp.float32),
          jax.ShapeDtypeStruct((N, 256), jnp.float32),
      ],
  )(agg3, h2, txt_emb, inv, wr3, b3, w4a, w4b, b4)


def kernel(node_x, text_x, edge_index,
           Wl1, Wr1, b1, Wl2, Wr2, b2, Wl3, Wr3, b3,
           Wt1, bt1, Wt2, bt2, W4, b4):
  src = edge_index[0].astype(jnp.int32)
  dst = edge_index[1].astype(jnp.int32)
  src_p = jnp.concatenate([src, jnp.zeros((E_PAD - E,), jnp.int32)])
  dst_p = jnp.concatenate([dst, jnp.full((E_PAD - E,), PAD_DST, jnp.int32)])

  zeros128 = jnp.zeros((N_ACC, 128), jnp.float32)
  ones128 = jnp.ones((B_E, 128), jnp.float32)

  b1r = b1.reshape(1, -1)
  b2r = b2.reshape(1, -1)
  b3r = b3.reshape(1, -1)
  bt1r = bt1.reshape(1, -1)
  bt2r = bt2.reshape(1, -1)
  b4r = b4.reshape(1, -1)
  w4a = W4[:128]
  w4b = W4[128:]

  agg1 = _seg_l1(node_x, src_p, dst_p, zeros128, ones128)
  txt_emb = _text_call(text_x, Wt1, bt1r, Wt2, bt2r)
  h1h, inv = _ta_call(agg1[:, :N, :], node_x, Wl1, Wr1, b1r)
  agg2 = _seg_l2(h1h, src_p, dst_p, zeros128, ones128)
  h2, z3 = _tb_call(agg2[:, :N, :], h1h, inv, Wl2, Wr2, b2r, Wl3)
  agg3 = _seg_l3(z3, src_p, dst_p, zeros128, ones128)
  lbl, final = _tc_call(agg3[:, :N, :], h2, txt_emb, inv, Wr3, b3r,
                        w4a, w4b, b4r)
  return (final, lbl, txt_emb)
